# Initial kernel scaffold; baseline (speedup 1.0000x reference)
import jax
import jax.numpy as jnp
from jax.experimental import pallas as pl


def kernel(x, edge_index, batch, W1_rel, W1_root, b1, Wg, att_src, att_dst, bg, W2_rel, W2_root, b2):
    raise NotImplementedError("write your pallas kernel here")


# speedup vs baseline: 6.1644x; 6.1644x over previous
import functools

import jax
import jax.numpy as jnp
from jax import lax
from jax.experimental import pallas as pl
from jax.experimental.pallas import tpu as pltpu

N = 10000
E = 160000
IN = 256
HID = 512
OUT = 256
HEADS = 2
NG = 64

_BR = 1000


def _mm_body(a_ref, b_ref, o_ref):
    o_ref[...] = jnp.dot(a_ref[...], b_ref[...],
                         preferred_element_type=jnp.float32)


def _matmul(a, b):
    n, k = a.shape
    k2, m = b.shape
    grid = n // _BR
    return pl.pallas_call(
        _mm_body,
        grid=(grid,),
        in_specs=[
            pl.BlockSpec((_BR, k), lambda i: (i, 0)),
            pl.BlockSpec((k2, m), lambda i: (0, 0)),
        ],
        out_specs=pl.BlockSpec((_BR, m), lambda i: (i, 0)),
        out_shape=jax.ShapeDtypeStruct((n, m), jnp.float32),
    )(a, b)


def kernel(x, edge_index, batch, W1_rel, W1_root, b1, Wg, att_src, att_dst,
           bg, W2_rel, W2_root, b2):
    src = edge_index[0]
    dst = edge_index[1]

    mu = jnp.mean(x, axis=0, keepdims=True)
    sd = jnp.std(x, axis=0, keepdims=True)
    xs = (x - mu) / jnp.where(sd == 0, 1.0, sd)
    xs = xs.astype(jnp.float32)

    Wg3 = Wg.reshape(IN, HEADS, HID)
    v_src = jnp.einsum("ihk,hk->ih", Wg3, att_src)
    v_dst = jnp.einsum("ihk,hk->ih", Wg3, att_dst)
    a_src = xs @ v_src
    a_dst = xs @ v_dst

    e_log = a_src[src] + a_dst[dst]
    e_log = jnp.where(e_log >= 0, e_log, 0.2 * e_log)
    ex = jnp.exp(e_log)
    e_self = a_src + a_dst
    e_self = jnp.where(e_self >= 0, e_self, 0.2 * e_self)
    ex_self = jnp.exp(e_self)

    denom = jax.ops.segment_sum(ex, dst, num_segments=N) + ex_self

    g = xs[src]
    agg1 = jax.ops.segment_sum(g, dst, num_segments=N)
    s0 = jax.ops.segment_sum(g * ex[:, 0:1], dst, num_segments=N)
    s1 = jax.ops.segment_sum(g * ex[:, 1:2], dst, num_segments=N)
    s0 = (s0 + ex_self[:, 0:1] * xs) / denom[:, 0:1]
    s1 = (s1 + ex_self[:, 1:2] * xs) / denom[:, 1:2]

    x1 = jax.nn.relu(_matmul(agg1, W1_rel) + _matmul(xs, W1_root) + b1)

    og0 = _matmul(s0, Wg3[:, 0, :])
    og1 = _matmul(s1, Wg3[:, 1, :])
    x_gat = jax.nn.relu(jnp.concatenate([og0, og1], axis=1) + bg)

    x2 = jnp.concatenate([x1, x_gat], axis=1)
    y2 = _matmul(x2, W2_rel)
    agg2 = jax.ops.segment_sum(y2[src], dst, num_segments=N)
    x3 = agg2 + _matmul(x2, W2_root) + b2

    sums = jax.ops.segment_sum(x3, batch, num_segments=NG)
    cnts = jax.ops.segment_sum(jnp.ones((N,), jnp.float32), batch,
                               num_segments=NG)
    pooled = sums / jnp.clip(cnts, 1.0)[:, None]
    return pooled
